# baseline (speedup 1.0000x reference)
import functools

import jax
import jax.numpy as jnp
import numpy as np
from jax import lax
from jax.experimental import pallas as pl
from jax.experimental.pallas import tpu as pltpu
from jax.experimental.pallas import tpu_sc as plsc

N_NODES = 50000
NEIGH = 16
H = 128
NPAD = 51200
BM = 3200
NBLK = NPAD // BM


def _row_mask(i):
    rows = lax.broadcasted_iota(jnp.int32, (BM, 1), 0) + i * BM
    return rows < N_NODES


def _clamp(i):
    return (jnp.minimum(i, (N_NODES - 1) // BM), 0)


def _prep_edge_call(bond_fea, rep, fbt):
    def body(b_ref, r_ref, f_ref, e_ref):
        i = pl.program_id(0)
        brep = jnp.dot(b_ref[...], r_ref[...],
                       preferred_element_type=jnp.float32,
                       precision=lax.Precision.HIGHEST)
        e_ref[...] = jnp.where(_row_mask(i),
                               jnp.exp(-((brep - f_ref[...]) ** 2)), 0.0)

    return pl.pallas_call(
        body,
        grid=(NBLK,),
        in_specs=[
            pl.BlockSpec((BM, NEIGH), _clamp),
            pl.BlockSpec((NEIGH, H), lambda i: (0, 0)),
            pl.BlockSpec((1, H), lambda i: (0, 0)),
        ],
        out_specs=pl.BlockSpec((BM, H), lambda i: (i, 0)),
        out_shape=jax.ShapeDtypeStruct((NPAD, H), jnp.float32),
    )(bond_fea, rep, fbt)


def _prep_ang_call(ang_flat):
    def body(a_ref, lo_ref, hi_ref):
        i = pl.program_id(0)
        m = _row_mask(i)
        g = jnp.exp(-((a_ref[...] + 1.0) ** 2) * 0.25)
        g = jnp.where(m, g, 0.0)
        lo_ref[...] = g[:, :H]
        hi_ref[...] = g[:, H:]

    return pl.pallas_call(
        body,
        grid=(NBLK,),
        in_specs=[pl.BlockSpec((BM, 256), _clamp)],
        out_specs=[
            pl.BlockSpec((BM, H), lambda i: (i, 0)),
            pl.BlockSpec((BM, H), lambda i: (i, 0)),
        ],
        out_shape=[
            jax.ShapeDtypeStruct((NPAD, H), jnp.float32),
            jax.ShapeDtypeStruct((NPAD, H), jnp.float32),
        ],
    )(ang_flat)


def _gin_mlp_call(zs, W1, b1, W2, b2):
    nz = len(zs)
    fin = H * nz

    def body(*refs):
        z_refs, (w1_ref, b1_ref, w2_ref, b2_ref, y_ref, s_ref) = \
            refs[:nz], refs[nz:]
        i = pl.program_id(0)
        z = (z_refs[0][...] if nz == 1 else
             jnp.concatenate([r[...] for r in z_refs], axis=1))
        h = jnp.maximum(
            jnp.dot(z, w1_ref[...],
                    preferred_element_type=jnp.float32) + b1_ref[...], 0.0)
        y = jnp.maximum(
            jnp.dot(h, w2_ref[...],
                    preferred_element_type=jnp.float32) + b2_ref[...], 0.0)
        y_ref[...] = y
        ym = jnp.where(_row_mask(i), y, 0.0)
        part = jnp.concatenate(
            [jnp.sum(ym, axis=0, keepdims=True),
             jnp.sum(ym * ym, axis=0, keepdims=True)], axis=0)

        @pl.when(i == 0)
        def _():
            s_ref[...] = part

        @pl.when(i > 0)
        def _():
            s_ref[...] += part

    return pl.pallas_call(
        body,
        grid=(NBLK,),
        in_specs=[pl.BlockSpec((BM, H), lambda i: (i, 0))] * nz + [
            pl.BlockSpec((fin, H), lambda i: (0, 0)),
            pl.BlockSpec((1, H), lambda i: (0, 0)),
            pl.BlockSpec((H, H), lambda i: (0, 0)),
            pl.BlockSpec((1, H), lambda i: (0, 0)),
        ],
        out_specs=[
            pl.BlockSpec((BM, H), lambda i: (i, 0)),
            pl.BlockSpec((2, H), lambda i: (0, 0)),
        ],
        out_shape=[
            jax.ShapeDtypeStruct((NPAD, H), jnp.float32),
            jax.ShapeDtypeStruct((2, H), jnp.float32),
        ],
    )(*zs, W1, b1.reshape(1, H), W2, b2.reshape(1, H))


def _bn_apply_call(y, stats, g, b):
    def body(y_ref, s_ref, g_ref, b_ref, o_ref):
        i = pl.program_id(0)
        m = s_ref[0:1, :] * (1.0 / N_NODES)
        v = s_ref[1:2, :] * (1.0 / N_NODES) - m * m
        scale = g_ref[...] * lax.rsqrt(v + 1e-5)
        x = (y_ref[...] - m) * scale + b_ref[...]
        o_ref[...] = jnp.where(_row_mask(i), x, 0.0)

    return pl.pallas_call(
        body,
        grid=(NBLK,),
        in_specs=[
            pl.BlockSpec((BM, H), lambda i: (i, 0)),
            pl.BlockSpec((2, H), lambda i: (0, 0)),
            pl.BlockSpec((1, H), lambda i: (0, 0)),
            pl.BlockSpec((1, H), lambda i: (0, 0)),
        ],
        out_specs=pl.BlockSpec((BM, H), lambda i: (i, 0)),
        out_shape=jax.ShapeDtypeStruct((NPAD, H), jnp.float32),
    )(y, stats, g.reshape(1, H), b.reshape(1, H))


def _head_call(yb, sb, ya, sa, p):
    h2, h4 = H // 2, H // 4

    def body(yb_ref, sb_ref, ya_ref, sa_ref, gb_ref, bb_ref, ga_ref, ba_ref,
             f1b, f1bb, f1a, f1ab, f2b, f2bb, f2a, f2ab, mw, mb, fw, fb_,
             o_ref):
        def bn(y_ref, s_ref, g_ref, be_ref):
            m = s_ref[0:1, :] * (1.0 / N_NODES)
            v = s_ref[1:2, :] * (1.0 / N_NODES) - m * m
            return (y_ref[...] - m) * (g_ref[...] * lax.rsqrt(v + 1e-5)) \
                + be_ref[...]

        dot = lambda a, b: jnp.dot(a, b, preferred_element_type=jnp.float32)
        bv = bn(yb_ref, sb_ref, gb_ref, bb_ref)
        bv = dot(bv, f1b[...]) + f1bb[...]
        bv = dot(bv, f2b[...]) + f2bb[...]
        av = bn(ya_ref, sa_ref, ga_ref, ba_ref)
        av = dot(av, f1a[...]) + f1ab[...]
        av = dot(av, f2a[...]) + f2ab[...]
        c = jnp.concatenate([bv, av], axis=1)
        c = dot(c, mw[...]) + mb[...]
        o_ref[...] = dot(c, fw[...]) + fb_[...]

    full = lambda r, c: pl.BlockSpec((r, c), lambda i: (0, 0))
    return pl.pallas_call(
        body,
        grid=(NBLK,),
        in_specs=[
            pl.BlockSpec((BM, H), lambda i: (i, 0)), full(2, H),
            pl.BlockSpec((BM, H), lambda i: (i, 0)), full(2, H),
            full(1, H), full(1, H), full(1, H), full(1, H),
            full(H, h2), full(1, h2), full(H, h2), full(1, h2),
            full(h2, h4), full(1, h4), full(h2, h4), full(1, h4),
            full(h2, h2), full(1, h2), full(h2, 2), full(1, 2),
        ],
        out_specs=pl.BlockSpec((BM, 2), lambda i: (i, 0)),
        out_shape=jax.ShapeDtypeStruct((N_NODES, 2), jnp.float32),
    )(yb, sb, ya, sa,
      p['eb_g'].reshape(1, H), p['eb_bb'].reshape(1, H),
      p['ea_g'].reshape(1, H), p['ea_bb'].reshape(1, H),
      p['f1b_W'], p['f1b_b'].reshape(1, h2),
      p['f1a_W'], p['f1a_b'].reshape(1, h2),
      p['f2b_W'], p['f2b_b'].reshape(1, h4),
      p['f2a_W'], p['f2a_b'].reshape(1, h4),
      p['mlp_W'], p['mlp_b'].reshape(1, h2),
      p['fc_W'], p['fc_b'].reshape(1, 2))


TPT = NPAD // 16
SUB = 128
NSUB = TPT // SUB
FC = 16


def _sc_agg_call(xs, idx_hbm):
    nz = len(xs)
    nch_core = 4 * nz
    mesh = plsc.VectorSubcoreMesh(core_axis_name="c", subcore_axis_name="s")

    @functools.partial(
        pl.kernel,
        mesh=mesh,
        out_type=[jax.ShapeDtypeStruct((NPAD, H), jnp.float32)] * nz,
        compiler_params=pltpu.CompilerParams(use_tc_tiling_on_sc=False),
        scratch_types=[
            pltpu.VMEM((NEIGH, SUB), jnp.int32),
            pltpu.VMEM((NEIGH, SUB), jnp.int32),
            pltpu.VMEM((SUB, FC), jnp.float32),
            pltpu.VMEM((SUB, FC), jnp.float32),
            pltpu.VMEM_SHARED((NPAD, FC), jnp.float32),
            pltpu.SemaphoreType.DMA,
            pltpu.SemaphoreType.DMA,
            pltpu.SemaphoreType.DMA,
        ],
    )
    def k(*refs):
        x_hbms = refs[:nz]
        idx_h = refs[nz]
        out_hbms = refs[nz + 1:nz + 1 + nz]
        idx0, idx1, xs0, xs1, acc_sh, sem0, sem1, sem_sc = refs[nz + 1 + nz:]
        cid = lax.axis_index("c")
        sid = lax.axis_index("s")
        r0 = sid * TPT
        for ci in range(nch_core):
            x_hbm = x_hbms[ci // 4]
            out_hbm = out_hbms[ci // 4]
            col0 = ((2 * ci) % 8 + cid) * FC

            def start_load(s, idx_v, xs_v, sem):
                pltpu.async_copy(idx_h.at[sid, s], idx_v, sem)
                pltpu.async_copy(
                    x_hbm.at[pl.ds(r0 + s * SUB, SUB), pl.ds(col0, FC)],
                    xs_v, sem)

            def wait_load(s, idx_v, xs_v, sem):
                pltpu.make_async_copy(idx_h.at[sid, s], idx_v, sem).wait()
                pltpu.make_async_copy(
                    x_hbm.at[pl.ds(r0 + s * SUB, SUB), pl.ds(col0, FC)],
                    xs_v, sem).wait()

            def scatter(idx_v, xs_v):
                cps = [
                    pltpu.async_copy(
                        xs_v, acc_sh.at[idx_v.at[j]], sem_sc, add=True)
                    for j in range(NEIGH)
                ]
                for c in cps:
                    c.wait()

            pltpu.sync_copy(x_hbm.at[pl.ds(r0, TPT), pl.ds(col0, FC)],
                            acc_sh.at[pl.ds(r0, TPT)])
            plsc.subcore_barrier()

            start_load(0, idx0, xs0, sem0)

            def body2(kk, carry):
                s0 = 2 * kk
                start_load(s0 + 1, idx1, xs1, sem1)
                wait_load(s0, idx0, xs0, sem0)
                scatter(idx0, xs0)
                start_load(s0 + 2, idx0, xs0, sem0)
                wait_load(s0 + 1, idx1, xs1, sem1)
                scatter(idx1, xs1)
                return carry

            lax.fori_loop(0, (NSUB - 1) // 2, body2, 0)
            wait_load(NSUB - 1, idx0, xs0, sem0)
            scatter(idx0, xs0)

            plsc.subcore_barrier()
            pltpu.sync_copy(acc_sh.at[pl.ds(r0, TPT)],
                            out_hbm.at[pl.ds(r0, TPT), pl.ds(col0, FC)])
            plsc.subcore_barrier()

    return k(*xs, idx_hbm)


def _build_idx(nbr_idx):
    pad_n = NPAD - N_NODES
    pad_dst = (jnp.arange(pad_n * NEIGH, dtype=nbr_idx.dtype) % N_NODES
               ).reshape(pad_n, NEIGH)
    nbr_p = jnp.concatenate([nbr_idx, pad_dst], axis=0)
    idx = nbr_p.reshape(16, NSUB, SUB, NEIGH)
    idx = jnp.transpose(idx, (0, 1, 3, 2))
    return idx.astype(jnp.int32)


def kernel(bond_fea, angle_fea, species, nbr_idx, crys_idx, params):
    p = params
    idx_hbm = _build_idx(nbr_idx)
    ang_flat = angle_fea.reshape(N_NODES, NEIGH * NEIGH)

    rep = jnp.asarray(np.repeat(np.eye(NEIGH, dtype=np.float32), 8, axis=1))
    fb = np.linspace(0.0, 8.0, 8, dtype=np.float32)
    fbt = jnp.asarray(np.tile(fb, NEIGH)[None, :])

    edge_p = _prep_edge_call(bond_fea, rep, fbt)
    zb, = _sc_agg_call([edge_p], idx_hbm)

    ang_lo, ang_hi = _prep_ang_call(ang_flat)
    za_lo, za_hi = _sc_agg_call([ang_lo, ang_hi], idx_hbm)

    yb, sb = _gin_mlp_call([zb], p['nb_W1'], p['nb_b1'],
                           p['nb_W2'], p['nb_b2'])
    b1 = _bn_apply_call(yb, sb, p['bnb_g'], p['bnb_b'])
    zb2, = _sc_agg_call([b1], idx_hbm)

    ya, sa = _gin_mlp_call([za_lo, za_hi], p['na_W1'], p['na_b1'],
                           p['na_W2'], p['na_b2'])
    a1 = _bn_apply_call(ya, sa, p['bna_g'], p['bna_b'])
    za2, = _sc_agg_call([a1], idx_hbm)

    yb2, sb2 = _gin_mlp_call([zb2], p['eb_W1'], p['eb_b1'],
                             p['eb_W2'], p['eb_b2'])
    ya2, sa2 = _gin_mlp_call([za2], p['ea_W1'], p['ea_b1'],
                             p['ea_W2'], p['ea_b2'])

    return _head_call(yb2, sb2, ya2, sa2, params)


# speedup vs baseline: 1.3280x; 1.3280x over previous
import functools

import jax
import jax.numpy as jnp
import numpy as np
from jax import lax
from jax.experimental import pallas as pl
from jax.experimental.pallas import tpu as pltpu
from jax.experimental.pallas import tpu_sc as plsc

N_NODES = 50000
NEIGH = 16
H = 128
NPAD = 51200
BM = 3200
NBLK = NPAD // BM


def _row_mask(i):
    rows = lax.broadcasted_iota(jnp.int32, (BM, 1), 0) + i * BM
    return rows < N_NODES


def _clamp(i):
    return (jnp.minimum(i, (N_NODES - 1) // BM), 0)


def _prep_edge_call(bond_fea, rep, fbt):
    def body(b_ref, r_ref, f_ref, e_ref):
        i = pl.program_id(0)
        brep = jnp.dot(b_ref[...], r_ref[...],
                       preferred_element_type=jnp.float32,
                       precision=lax.Precision.HIGHEST)
        e_ref[...] = jnp.where(_row_mask(i),
                               jnp.exp(-((brep - f_ref[...]) ** 2)), 0.0)

    return pl.pallas_call(
        body,
        grid=(NBLK,),
        in_specs=[
            pl.BlockSpec((BM, NEIGH), _clamp),
            pl.BlockSpec((NEIGH, H), lambda i: (0, 0)),
            pl.BlockSpec((1, H), lambda i: (0, 0)),
        ],
        out_specs=pl.BlockSpec((BM, H), lambda i: (i, 0)),
        out_shape=jax.ShapeDtypeStruct((NPAD, H), jnp.float32),
    )(bond_fea, rep, fbt)


def _prep_ang_call(ang_flat):
    def body(a_ref, lo_ref, hi_ref):
        i = pl.program_id(0)
        m = _row_mask(i)
        g = jnp.exp(-((a_ref[...] + 1.0) ** 2) * 0.25)
        g = jnp.where(m, g, 0.0)
        lo_ref[...] = g[:, :H]
        hi_ref[...] = g[:, H:]

    return pl.pallas_call(
        body,
        grid=(NBLK,),
        in_specs=[pl.BlockSpec((BM, 256), _clamp)],
        out_specs=[
            pl.BlockSpec((BM, H), lambda i: (i, 0)),
            pl.BlockSpec((BM, H), lambda i: (i, 0)),
        ],
        out_shape=[
            jax.ShapeDtypeStruct((NPAD, H), jnp.float32),
            jax.ShapeDtypeStruct((NPAD, H), jnp.float32),
        ],
    )(ang_flat)


def _gin_mlp_call(zs, W1, b1, W2, b2):
    nz = len(zs)
    fin = H * nz

    def body(*refs):
        z_refs, (w1_ref, b1_ref, w2_ref, b2_ref, y_ref, s_ref) = \
            refs[:nz], refs[nz:]
        i = pl.program_id(0)
        z = (z_refs[0][...] if nz == 1 else
             jnp.concatenate([r[...] for r in z_refs], axis=1))
        h = jnp.maximum(
            jnp.dot(z, w1_ref[...],
                    preferred_element_type=jnp.float32) + b1_ref[...], 0.0)
        y = jnp.maximum(
            jnp.dot(h, w2_ref[...],
                    preferred_element_type=jnp.float32) + b2_ref[...], 0.0)
        y_ref[...] = y
        ym = jnp.where(_row_mask(i), y, 0.0)
        part = jnp.concatenate(
            [jnp.sum(ym, axis=0, keepdims=True),
             jnp.sum(ym * ym, axis=0, keepdims=True)], axis=0)

        @pl.when(i == 0)
        def _():
            s_ref[...] = part

        @pl.when(i > 0)
        def _():
            s_ref[...] += part

    return pl.pallas_call(
        body,
        grid=(NBLK,),
        in_specs=[pl.BlockSpec((BM, H), lambda i: (i, 0))] * nz + [
            pl.BlockSpec((fin, H), lambda i: (0, 0)),
            pl.BlockSpec((1, H), lambda i: (0, 0)),
            pl.BlockSpec((H, H), lambda i: (0, 0)),
            pl.BlockSpec((1, H), lambda i: (0, 0)),
        ],
        out_specs=[
            pl.BlockSpec((BM, H), lambda i: (i, 0)),
            pl.BlockSpec((2, H), lambda i: (0, 0)),
        ],
        out_shape=[
            jax.ShapeDtypeStruct((NPAD, H), jnp.float32),
            jax.ShapeDtypeStruct((2, H), jnp.float32),
        ],
    )(*zs, W1, b1.reshape(1, H), W2, b2.reshape(1, H))


def _bn_apply_call(y, stats, g, b):
    def body(y_ref, s_ref, g_ref, b_ref, o_ref):
        i = pl.program_id(0)
        m = s_ref[0:1, :] * (1.0 / N_NODES)
        v = s_ref[1:2, :] * (1.0 / N_NODES) - m * m
        scale = g_ref[...] * lax.rsqrt(v + 1e-5)
        x = (y_ref[...] - m) * scale + b_ref[...]
        o_ref[...] = jnp.where(_row_mask(i), x, 0.0)

    return pl.pallas_call(
        body,
        grid=(NBLK,),
        in_specs=[
            pl.BlockSpec((BM, H), lambda i: (i, 0)),
            pl.BlockSpec((2, H), lambda i: (0, 0)),
            pl.BlockSpec((1, H), lambda i: (0, 0)),
            pl.BlockSpec((1, H), lambda i: (0, 0)),
        ],
        out_specs=pl.BlockSpec((BM, H), lambda i: (i, 0)),
        out_shape=jax.ShapeDtypeStruct((NPAD, H), jnp.float32),
    )(y, stats, g.reshape(1, H), b.reshape(1, H))


def _head_call(yb, sb, ya, sa, p):
    h2, h4 = H // 2, H // 4

    def body(yb_ref, sb_ref, ya_ref, sa_ref, gb_ref, bb_ref, ga_ref, ba_ref,
             f1b, f1bb, f1a, f1ab, f2b, f2bb, f2a, f2ab, mw, mb, fw, fb_,
             o_ref):
        def bn(y_ref, s_ref, g_ref, be_ref):
            m = s_ref[0:1, :] * (1.0 / N_NODES)
            v = s_ref[1:2, :] * (1.0 / N_NODES) - m * m
            return (y_ref[...] - m) * (g_ref[...] * lax.rsqrt(v + 1e-5)) \
                + be_ref[...]

        dot = lambda a, b: jnp.dot(a, b, preferred_element_type=jnp.float32)
        bv = bn(yb_ref, sb_ref, gb_ref, bb_ref)
        bv = dot(bv, f1b[...]) + f1bb[...]
        bv = dot(bv, f2b[...]) + f2bb[...]
        av = bn(ya_ref, sa_ref, ga_ref, ba_ref)
        av = dot(av, f1a[...]) + f1ab[...]
        av = dot(av, f2a[...]) + f2ab[...]
        c = jnp.concatenate([bv, av], axis=1)
        c = dot(c, mw[...]) + mb[...]
        o_ref[...] = dot(c, fw[...]) + fb_[...]

    full = lambda r, c: pl.BlockSpec((r, c), lambda i: (0, 0))
    return pl.pallas_call(
        body,
        grid=(NBLK,),
        in_specs=[
            pl.BlockSpec((BM, H), lambda i: (i, 0)), full(2, H),
            pl.BlockSpec((BM, H), lambda i: (i, 0)), full(2, H),
            full(1, H), full(1, H), full(1, H), full(1, H),
            full(H, h2), full(1, h2), full(H, h2), full(1, h2),
            full(h2, h4), full(1, h4), full(h2, h4), full(1, h4),
            full(h2, h2), full(1, h2), full(h2, 2), full(1, 2),
        ],
        out_specs=pl.BlockSpec((BM, 2), lambda i: (i, 0)),
        out_shape=jax.ShapeDtypeStruct((N_NODES, 2), jnp.float32),
    )(yb, sb, ya, sa,
      p['eb_g'].reshape(1, H), p['eb_bb'].reshape(1, H),
      p['ea_g'].reshape(1, H), p['ea_bb'].reshape(1, H),
      p['f1b_W'], p['f1b_b'].reshape(1, h2),
      p['f1a_W'], p['f1a_b'].reshape(1, h2),
      p['f2b_W'], p['f2b_b'].reshape(1, h4),
      p['f2a_W'], p['f2a_b'].reshape(1, h4),
      p['mlp_W'], p['mlp_b'].reshape(1, h2),
      p['fc_W'], p['fc_b'].reshape(1, 2))


TPT = NPAD // 16
SUB = 128
NSUB = TPT // SUB
FC = 32


def _sc_agg_call(xs, idx_hbm):
    nz = len(xs)
    nch_core = 2 * nz
    mesh = plsc.VectorSubcoreMesh(core_axis_name="c", subcore_axis_name="s")

    @functools.partial(
        pl.kernel,
        mesh=mesh,
        out_type=[jax.ShapeDtypeStruct((NPAD, H), jnp.float32)] * nz,
        compiler_params=pltpu.CompilerParams(use_tc_tiling_on_sc=False),
        scratch_types=[
            pltpu.VMEM((NEIGH, SUB), jnp.int32),
            pltpu.VMEM((NEIGH, SUB), jnp.int32),
            pltpu.VMEM((SUB, FC), jnp.float32),
            pltpu.VMEM((SUB, FC), jnp.float32),
            pltpu.VMEM_SHARED((NPAD, FC), jnp.float32),
            pltpu.SemaphoreType.DMA,
            pltpu.SemaphoreType.DMA,
            pltpu.SemaphoreType.DMA,
        ],
    )
    def k(*refs):
        x_hbms = refs[:nz]
        idx_h = refs[nz]
        out_hbms = refs[nz + 1:nz + 1 + nz]
        idx0, idx1, xs0, xs1, acc_sh, sem0, sem1, sem_sc = refs[nz + 1 + nz:]
        cid = lax.axis_index("c")
        sid = lax.axis_index("s")
        r0 = sid * TPT
        for ci in range(nch_core):
            x_hbm = x_hbms[ci // 2]
            out_hbm = out_hbms[ci // 2]
            col0 = ((2 * ci) % 4 + cid) * FC

            def start_load(s, idx_v, xs_v, sem):
                pltpu.async_copy(idx_h.at[sid, s], idx_v, sem)
                pltpu.async_copy(
                    x_hbm.at[pl.ds(r0 + s * SUB, SUB), pl.ds(col0, FC)],
                    xs_v, sem)

            def wait_load(s, idx_v, xs_v, sem):
                pltpu.make_async_copy(idx_h.at[sid, s], idx_v, sem).wait()
                pltpu.make_async_copy(
                    x_hbm.at[pl.ds(r0 + s * SUB, SUB), pl.ds(col0, FC)],
                    xs_v, sem).wait()

            def scatter(idx_v, xs_v):
                cps = [
                    pltpu.async_copy(
                        xs_v, acc_sh.at[idx_v.at[j]], sem_sc, add=True)
                    for j in range(NEIGH)
                ]
                for c in cps:
                    c.wait()

            pltpu.sync_copy(x_hbm.at[pl.ds(r0, TPT), pl.ds(col0, FC)],
                            acc_sh.at[pl.ds(r0, TPT)])
            plsc.subcore_barrier()

            start_load(0, idx0, xs0, sem0)

            def body2(kk, carry):
                s0 = 2 * kk
                start_load(s0 + 1, idx1, xs1, sem1)
                wait_load(s0, idx0, xs0, sem0)
                scatter(idx0, xs0)
                start_load(s0 + 2, idx0, xs0, sem0)
                wait_load(s0 + 1, idx1, xs1, sem1)
                scatter(idx1, xs1)
                return carry

            lax.fori_loop(0, (NSUB - 1) // 2, body2, 0)
            wait_load(NSUB - 1, idx0, xs0, sem0)
            scatter(idx0, xs0)

            plsc.subcore_barrier()
            pltpu.sync_copy(acc_sh.at[pl.ds(r0, TPT)],
                            out_hbm.at[pl.ds(r0, TPT), pl.ds(col0, FC)])
            plsc.subcore_barrier()

    return k(*xs, idx_hbm)


def _build_idx(nbr_idx):
    pad_n = NPAD - N_NODES
    pad_dst = (jnp.arange(pad_n * NEIGH, dtype=nbr_idx.dtype) % N_NODES
               ).reshape(pad_n, NEIGH)
    nbr_p = jnp.concatenate([nbr_idx, pad_dst], axis=0)
    idx = nbr_p.reshape(16, NSUB, SUB, NEIGH)
    idx = jnp.transpose(idx, (0, 1, 3, 2))
    return idx.astype(jnp.int32)


def kernel(bond_fea, angle_fea, species, nbr_idx, crys_idx, params):
    p = params
    idx_hbm = _build_idx(nbr_idx)
    ang_flat = angle_fea.reshape(N_NODES, NEIGH * NEIGH)

    rep = jnp.asarray(np.repeat(np.eye(NEIGH, dtype=np.float32), 8, axis=1))
    fb = np.linspace(0.0, 8.0, 8, dtype=np.float32)
    fbt = jnp.asarray(np.tile(fb, NEIGH)[None, :])

    edge_p = _prep_edge_call(bond_fea, rep, fbt)
    zb, = _sc_agg_call([edge_p], idx_hbm)

    ang_lo, ang_hi = _prep_ang_call(ang_flat)
    ang_lo, ang_hi, zb = lax.optimization_barrier((ang_lo, ang_hi, zb))
    za_lo, za_hi = _sc_agg_call([ang_lo, ang_hi], idx_hbm)

    yb, sb = _gin_mlp_call([zb], p['nb_W1'], p['nb_b1'],
                           p['nb_W2'], p['nb_b2'])
    b1 = _bn_apply_call(yb, sb, p['bnb_g'], p['bnb_b'])
    zb2, = _sc_agg_call([b1], idx_hbm)

    ya, sa = _gin_mlp_call([za_lo, za_hi], p['na_W1'], p['na_b1'],
                           p['na_W2'], p['na_b2'])
    a1 = _bn_apply_call(ya, sa, p['bna_g'], p['bna_b'])
    za2, = _sc_agg_call([a1], idx_hbm)

    yb2, sb2 = _gin_mlp_call([zb2], p['eb_W1'], p['eb_b1'],
                             p['eb_W2'], p['eb_b2'])
    ya2, sa2 = _gin_mlp_call([za2], p['ea_W1'], p['ea_b1'],
                             p['ea_W2'], p['ea_b2'])

    return _head_call(yb2, sb2, ya2, sa2, params)
